# Initial kernel scaffold; baseline (speedup 1.0000x reference)
import jax
import jax.numpy as jnp
from jax.experimental import pallas as pl


def kernel(x, edge_index, time_steps, W1, b1, W2, b2, Wt, bt, Wf, bf):
    raise NotImplementedError("write your pallas kernel here")


# speedup vs baseline: 12.2333x; 12.2333x over previous
import functools

import jax
import jax.numpy as jnp
from jax import lax
from jax.experimental import pallas as pl
from jax.experimental.pallas import tpu as pltpu
from jax.experimental.pallas import tpu_sc as plsc

N = 10000
NP = 10240
E = 320000
F = 128
H = 128
OUT = 2

NC = 2
NS = 16
NW = NC * NS
EPW = E // NW
CH = 40
NCH = EPW // CH
NBUF = 5
NOUTER = NCH // NBUF
RPT = NP // NS

_PREC = lax.Precision.HIGHEST


def _prop_body(src_hbm, dst_hbm, s_hbm, z_hbm, out_hbm, *scratch):
    idx_d = scratch[0:NBUF]
    rows = scratch[NBUF:2 * NBUF]
    src_v = scratch[2 * NBUF]
    shared = scratch[2 * NBUF + 1]
    gsem = scratch[2 * NBUF + 2:2 * NBUF + 2 + NBUF]
    isem = scratch[2 * NBUF + 2 + NBUF:2 * NBUF + 2 + 2 * NBUF]

    cid = lax.axis_index("c")
    sid = lax.axis_index("s")
    wid = cid * NS + sid
    base = pl.multiple_of(wid * EPW, EPW)

    pltpu.sync_copy(z_hbm, shared.at[pl.ds(sid * RPT, RPT)])
    pltpu.sync_copy(src_hbm.at[pl.ds(base, EPW)], src_v)
    plsc.subcore_barrier()

    def outer(o, carry):
        c0 = pl.multiple_of(o * NBUF * CH, NBUF * CH)
        gcp = []
        icp = []
        for b in range(NBUF):
            icp.append(pltpu.async_copy(
                dst_hbm.at[pl.ds(base + c0 + b * CH, CH)], idx_d[b], isem[b]))
            gcp.append(pltpu.async_copy(
                s_hbm.at[src_v.at[pl.ds(c0 + b * CH, CH)]], rows[b], gsem[b]))
        for b in range(NBUF):
            gcp[b].wait()
            icp[b].wait()
            pltpu.sync_copy(rows[b], shared.at[idx_d[b]], add=True)
        return carry

    lax.fori_loop(0, NOUTER, outer, 0, unroll=False)

    plsc.subcore_barrier()
    off = cid * NP + sid * RPT
    pltpu.sync_copy(shared.at[pl.ds(sid * RPT, RPT)], out_hbm.at[pl.ds(off, RPT)])


@functools.cache
def _sc_mesh():
    return plsc.VectorSubcoreMesh(core_axis_name="c", subcore_axis_name="s",
                                  num_cores=NC, num_subcores=NS)


@functools.cache
def _make_prop():
    return pl.kernel(
        _prop_body,
        out_type=jax.ShapeDtypeStruct((NC * NP, F), jnp.float32),
        mesh=_sc_mesh(),
        scratch_types=(
            [pltpu.VMEM((CH,), jnp.int32) for _ in range(NBUF)]
            + [pltpu.VMEM((CH, F), jnp.float32) for _ in range(NBUF)]
            + [pltpu.VMEM((EPW,), jnp.int32)]
            + [pltpu.VMEM_SHARED((NP, F), jnp.float32)]
            + [pltpu.SemaphoreType.DMA for _ in range(2 * NBUF)]
        ),
    )


def _prop_call(src, dst, s, zeros):
    return _make_prop()(src, dst, s, zeros)


def _deg_body(dst_hbm, ones_hbm, z_hbm, out_hbm, *scratch):
    idx_d = scratch[0:NBUF]
    ones_v = scratch[NBUF]
    shared = scratch[NBUF + 1]
    isem = scratch[NBUF + 2:NBUF + 2 + NBUF]

    cid = lax.axis_index("c")
    sid = lax.axis_index("s")
    wid = cid * NS + sid
    base = pl.multiple_of(wid * EPW, EPW)

    pltpu.sync_copy(z_hbm, shared.at[pl.ds(sid * RPT, RPT)])
    pltpu.sync_copy(ones_hbm, ones_v)
    plsc.subcore_barrier()

    def outer(o, carry):
        c0 = pl.multiple_of(o * NBUF * CH, NBUF * CH)
        icp = []
        for b in range(NBUF):
            icp.append(pltpu.async_copy(
                dst_hbm.at[pl.ds(base + c0 + b * CH, CH)], idx_d[b], isem[b]))
        for b in range(NBUF):
            icp[b].wait()
            pltpu.sync_copy(ones_v, shared.at[idx_d[b]], add=True)
        return carry

    lax.fori_loop(0, NOUTER, outer, 0, unroll=False)

    plsc.subcore_barrier()
    off = cid * NP + sid * RPT
    pltpu.sync_copy(shared.at[pl.ds(sid * RPT, RPT)], out_hbm.at[pl.ds(off, RPT)])


@functools.cache
def _make_deg():
    return pl.kernel(
        _deg_body,
        out_type=jax.ShapeDtypeStruct((NC * NP, F), jnp.float32),
        mesh=_sc_mesh(),
        scratch_types=(
            [pltpu.VMEM((CH,), jnp.int32) for _ in range(NBUF)]
            + [pltpu.VMEM((CH, F), jnp.float32)]
            + [pltpu.VMEM_SHARED((NP, F), jnp.float32)]
            + [pltpu.SemaphoreType.DMA for _ in range(NBUF)]
        ),
    )


def _deg_call(dst, ones, zeros):
    return _make_deg()(dst, ones, zeros)


def _prep_body(degp_ref, ts_ref, x_ref, dinv_ref, tmm_ref, xs_ref):
    deg = degp_ref[0, :, 0:1] + degp_ref[1, :, 0:1]
    dinv = jnp.where(deg > 0.0, lax.rsqrt(deg), 0.0)
    dinv_ref[...] = dinv
    ts = ts_ref[...]
    tmm_ref[...] = jnp.concatenate(
        [jnp.min(ts).reshape(1, 1), jnp.max(ts).reshape(1, 1)], axis=1)
    xs_ref[...] = x_ref[...] * dinv


def _prep_call(degp, ts_r, x):
    return pl.pallas_call(
        _prep_body,
        out_shape=(
            jax.ShapeDtypeStruct((NP, 1), jnp.float32),
            jax.ShapeDtypeStruct((1, 2), jnp.float32),
            jax.ShapeDtypeStruct((NP, F), jnp.float32),
        ),
    )(degp, ts_r, x)


_BLK = 512
_GRID = NP // _BLK


def _row_spec(r, c):
    return pl.BlockSpec((r, c), lambda i: (i, 0))


def _part_spec(c):
    return pl.BlockSpec((NC, _BLK, c), lambda i: (0, i, 0))


def _full_spec(r, c):
    return pl.BlockSpec((r, c), lambda i: (0, 0))


def _scale_body(up_ref, dinv_ref, s_ref):
    d = dinv_ref[...]
    s_ref[...] = (up_ref[0] + up_ref[1]) * (d * d)


def _scale_call(up, dinv):
    return pl.pallas_call(
        _scale_body,
        grid=(_GRID,),
        in_specs=[_part_spec(F), _row_spec(_BLK, 1)],
        out_specs=_row_spec(_BLK, F),
        out_shape=jax.ShapeDtypeStruct((NP, F), jnp.float32),
    )(up, dinv)


def _mm(a, w):
    return jnp.dot(a, w, preferred_element_type=jnp.float32, precision=_PREC)


def _l1_body(x_ref, u1_ref, u2_ref, u3_ref, dinv_ref, w1_ref, b1_ref,
             g_ref, gs_ref):
    d = dinv_ref[...]
    h1 = (u1_ref[0] + u1_ref[1]) * d
    h2 = (u2_ref[0] + u2_ref[1]) * d
    h3 = (u3_ref[0] + u3_ref[1]) * d
    acc = _mm(x_ref[...], w1_ref[0:F, :])
    acc = acc + _mm(h1, w1_ref[F:2 * F, :])
    acc = acc + _mm(h2, w1_ref[2 * F:3 * F, :])
    acc = acc + _mm(h3, w1_ref[3 * F:4 * F, :])
    g = jnp.maximum(acc + b1_ref[...], 0.0)
    g_ref[...] = g
    gs_ref[...] = g * d


def _l1_call(x, u1, u2, u3, dinv, w1, b1):
    return pl.pallas_call(
        _l1_body,
        grid=(_GRID,),
        in_specs=[_row_spec(_BLK, F), _part_spec(F), _part_spec(F),
                  _part_spec(F), _row_spec(_BLK, 1),
                  _full_spec(4 * F, H), _full_spec(1, H)],
        out_specs=(_row_spec(_BLK, H), _row_spec(_BLK, H)),
        out_shape=(jax.ShapeDtypeStruct((NP, H), jnp.float32),
                   jax.ShapeDtypeStruct((NP, H), jnp.float32)),
    )(x, u1, u2, u3, dinv, w1, b1)


def _l2_body(g_ref, v1_ref, v2_ref, dinv_ref, ts_ref, tmm_ref, w2_ref, b2_ref,
             wt_ref, bt_ref, wf_ref, bf_ref, out_ref):
    d = dinv_ref[...]
    h1 = (v1_ref[0] + v1_ref[1]) * d
    h2 = (v2_ref[0] + v2_ref[1]) * d
    acc = _mm(g_ref[...], w2_ref[0:H, :])
    acc = acc + _mm(h1, w2_ref[H:2 * H, :])
    acc = acc + _mm(h2, w2_ref[2 * H:3 * H, :])
    z = jnp.maximum(acc + b2_ref[...], 0.0)
    tmin = tmm_ref[0, 0]
    tmax = tmm_ref[0, 1]
    tn = (ts_ref[...] - tmin) / (tmax - tmin + 1e-8)
    te = tn * wt_ref[...] + bt_ref[...]
    out = _mm(z, wf_ref[0:H, :]) + _mm(te, wf_ref[H:2 * H, :]) + bf_ref[...]
    out_ref[...] = out


def _l2_call(g, v1, v2, dinv, ts2, tmm, w2, b2, wt, bt, wf, bf):
    return pl.pallas_call(
        _l2_body,
        grid=(_GRID,),
        in_specs=[_row_spec(_BLK, H), _part_spec(H), _part_spec(H),
                  _row_spec(_BLK, 1), _row_spec(_BLK, 1), _full_spec(1, 2),
                  _full_spec(3 * H, H), _full_spec(1, H),
                  _full_spec(1, H), _full_spec(1, H),
                  _full_spec(2 * H, OUT), _full_spec(1, OUT)],
        out_specs=_row_spec(_BLK, OUT),
        out_shape=jax.ShapeDtypeStruct((NP, OUT), jnp.float32),
    )(g, v1, v2, dinv, ts2, tmm, w2, b2, wt, bt, wf, bf)


def kernel(x, edge_index, time_steps, W1, b1, W2, b2, Wt, bt, Wf, bf):
    src = edge_index[0]
    dst = edge_index[1]
    ones_ch = jnp.ones((CH, F), jnp.float32)
    zeros128 = jnp.zeros((RPT, F), jnp.float32)

    xp = jnp.pad(x, ((0, NP - N), (0, 0)))
    degp = _deg_call(dst, ones_ch, zeros128).reshape(NC, NP, F)
    ts_r = time_steps.astype(jnp.float32).reshape(625, 16)
    dinv, tmm, xs = _prep_call(degp, ts_r, xp)

    u1 = _prop_call(src, dst, xs, zeros128).reshape(NC, NP, F)
    s1 = _scale_call(u1, dinv)
    u2 = _prop_call(src, dst, s1, zeros128).reshape(NC, NP, F)
    s2 = _scale_call(u2, dinv)
    u3 = _prop_call(src, dst, s2, zeros128).reshape(NC, NP, F)

    g, gs = _l1_call(xp, u1, u2, u3, dinv, W1, b1.reshape(1, H))

    v1 = _prop_call(src, dst, gs, zeros128).reshape(NC, NP, F)
    t1 = _scale_call(v1, dinv)
    v2 = _prop_call(src, dst, t1, zeros128).reshape(NC, NP, F)

    ts2 = jnp.pad(time_steps.astype(jnp.float32), (0, NP - N)).reshape(NP, 1)
    out = _l2_call(g, v1, v2, dinv, ts2, tmm, W2, b2.reshape(1, H),
                   Wt, bt.reshape(1, H), Wf, bf.reshape(1, OUT))
    return out[:N]


# speedup vs baseline: 16.4341x; 1.3434x over previous
import functools

import jax
import jax.numpy as jnp
from jax import lax
from jax.experimental import pallas as pl
from jax.experimental.pallas import tpu as pltpu
from jax.experimental.pallas import tpu_sc as plsc

N = 10000
NP = 10240
E = 320000
F = 128
H = 128
OUT = 2

NC = 2
NS = 16
NW = NC * NS
EPW = E // NW
CH = 40
NCH = EPW // CH
NBUF = 5
NOUTER = NCH // NBUF
RPT = NP // NS

_PREC = lax.Precision.HIGHEST


def _prop_body(src_hbm, dst_hbm, s_hbm, z_hbm, out_hbm, *scratch):
    idx_d = scratch[0:NBUF]
    rows = scratch[NBUF:2 * NBUF]
    src_v = scratch[2 * NBUF]
    shared = scratch[2 * NBUF + 1]
    gsem = scratch[2 * NBUF + 2:2 * NBUF + 2 + NBUF]
    isem = scratch[2 * NBUF + 2 + NBUF:2 * NBUF + 2 + 2 * NBUF]

    cid = lax.axis_index("c")
    sid = lax.axis_index("s")
    wid = cid * NS + sid
    base = pl.multiple_of(wid * EPW, EPW)

    pltpu.sync_copy(z_hbm, shared.at[pl.ds(sid * RPT, RPT)])
    pltpu.sync_copy(src_hbm.at[pl.ds(base, EPW)], src_v)
    plsc.subcore_barrier()

    def fire(c0, b):
        pltpu.async_copy(
            dst_hbm.at[pl.ds(base + c0 + b * CH, CH)], idx_d[b], isem[b])
        pltpu.async_copy(
            s_hbm.at[src_v.at[pl.ds(c0 + b * CH, CH)]], rows[b], gsem[b])

    for b in range(NBUF):
        fire(0, b)

    def outer(o, carry):
        c1 = pl.multiple_of((o + 1) * NBUF * CH, NBUF * CH)
        for b in range(NBUF):
            pltpu.make_async_copy(dst_hbm.at[pl.ds(0, CH)], idx_d[b],
                                  isem[b]).wait()
            pltpu.make_async_copy(s_hbm.at[pl.ds(0, CH)], rows[b],
                                  gsem[b]).wait()
            pltpu.sync_copy(rows[b], shared.at[idx_d[b]], add=True)

            @pl.when(o < NOUTER - 1)
            def _():
                fire(c1, b)

        return carry

    lax.fori_loop(0, NOUTER, outer, 0, unroll=False)

    plsc.subcore_barrier()
    off = cid * NP + sid * RPT
    pltpu.sync_copy(shared.at[pl.ds(sid * RPT, RPT)], out_hbm.at[pl.ds(off, RPT)])


@functools.cache
def _sc_mesh():
    return plsc.VectorSubcoreMesh(core_axis_name="c", subcore_axis_name="s",
                                  num_cores=NC, num_subcores=NS)


@functools.cache
def _make_prop():
    return pl.kernel(
        _prop_body,
        out_type=jax.ShapeDtypeStruct((NC * NP, F), jnp.float32),
        mesh=_sc_mesh(),
        scratch_types=(
            [pltpu.VMEM((CH,), jnp.int32) for _ in range(NBUF)]
            + [pltpu.VMEM((CH, F), jnp.float32) for _ in range(NBUF)]
            + [pltpu.VMEM((EPW,), jnp.int32)]
            + [pltpu.VMEM_SHARED((NP, F), jnp.float32)]
            + [pltpu.SemaphoreType.DMA for _ in range(2 * NBUF)]
        ),
    )


def _prop_call(src, dst, s, zeros):
    return _make_prop()(src, dst, s, zeros)


def _deg_body(dst_hbm, ones_hbm, z_hbm, out_hbm, *scratch):
    idx_d = scratch[0:NBUF]
    ones_v = scratch[NBUF]
    shared = scratch[NBUF + 1]
    isem = scratch[NBUF + 2:NBUF + 2 + NBUF]

    cid = lax.axis_index("c")
    sid = lax.axis_index("s")
    wid = cid * NS + sid
    base = pl.multiple_of(wid * EPW, EPW)

    pltpu.sync_copy(z_hbm, shared.at[pl.ds(sid * RPT, RPT)])
    pltpu.sync_copy(ones_hbm, ones_v)
    plsc.subcore_barrier()

    def fire(c0, b):
        pltpu.async_copy(
            dst_hbm.at[pl.ds(base + c0 + b * CH, CH)], idx_d[b], isem[b])

    for b in range(NBUF):
        fire(0, b)

    def outer(o, carry):
        c1 = pl.multiple_of((o + 1) * NBUF * CH, NBUF * CH)
        for b in range(NBUF):
            pltpu.make_async_copy(dst_hbm.at[pl.ds(0, CH)], idx_d[b],
                                  isem[b]).wait()
            pltpu.sync_copy(ones_v, shared.at[idx_d[b]], add=True)

            @pl.when(o < NOUTER - 1)
            def _():
                fire(c1, b)

        return carry

    lax.fori_loop(0, NOUTER, outer, 0, unroll=False)

    plsc.subcore_barrier()
    off = cid * NP + sid * RPT
    pltpu.sync_copy(shared.at[pl.ds(sid * RPT, RPT)], out_hbm.at[pl.ds(off, RPT)])


@functools.cache
def _make_deg():
    return pl.kernel(
        _deg_body,
        out_type=jax.ShapeDtypeStruct((NC * NP, F), jnp.float32),
        mesh=_sc_mesh(),
        scratch_types=(
            [pltpu.VMEM((CH,), jnp.int32) for _ in range(NBUF)]
            + [pltpu.VMEM((CH, F), jnp.float32)]
            + [pltpu.VMEM_SHARED((NP, F), jnp.float32)]
            + [pltpu.SemaphoreType.DMA for _ in range(NBUF)]
        ),
    )


def _deg_call(dst, ones, zeros):
    return _make_deg()(dst, ones, zeros)


def _prep_body(degp_ref, ts_ref, x_ref, dinv_ref, tmm_ref, xs_ref):
    deg = degp_ref[0, :, 0:1] + degp_ref[1, :, 0:1]
    dinv = jnp.where(deg > 0.0, lax.rsqrt(deg), 0.0)
    dinv_ref[...] = dinv
    ts = ts_ref[...]
    tmm_ref[...] = jnp.concatenate(
        [jnp.min(ts).reshape(1, 1), jnp.max(ts).reshape(1, 1)], axis=1)
    xs_ref[...] = x_ref[...] * dinv


def _prep_call(degp, ts_r, x):
    return pl.pallas_call(
        _prep_body,
        out_shape=(
            jax.ShapeDtypeStruct((NP, 1), jnp.float32),
            jax.ShapeDtypeStruct((1, 2), jnp.float32),
            jax.ShapeDtypeStruct((NP, F), jnp.float32),
        ),
    )(degp, ts_r, x)


_BLK = 512
_GRID = NP // _BLK


def _row_spec(r, c):
    return pl.BlockSpec((r, c), lambda i: (i, 0))


def _part_spec(c):
    return pl.BlockSpec((NC, _BLK, c), lambda i: (0, i, 0))


def _full_spec(r, c):
    return pl.BlockSpec((r, c), lambda i: (0, 0))


def _scale_body(up_ref, dinv_ref, s_ref):
    d = dinv_ref[...]
    s_ref[...] = (up_ref[0] + up_ref[1]) * (d * d)


def _scale_call(up, dinv):
    return pl.pallas_call(
        _scale_body,
        grid=(_GRID,),
        in_specs=[_part_spec(F), _row_spec(_BLK, 1)],
        out_specs=_row_spec(_BLK, F),
        out_shape=jax.ShapeDtypeStruct((NP, F), jnp.float32),
    )(up, dinv)


def _mm(a, w):
    return jnp.dot(a, w, preferred_element_type=jnp.float32, precision=_PREC)


def _l1_body(x_ref, u1_ref, u2_ref, u3_ref, dinv_ref, w1_ref, b1_ref,
             g_ref, gs_ref):
    d = dinv_ref[...]
    h1 = (u1_ref[0] + u1_ref[1]) * d
    h2 = (u2_ref[0] + u2_ref[1]) * d
    h3 = (u3_ref[0] + u3_ref[1]) * d
    acc = _mm(x_ref[...], w1_ref[0:F, :])
    acc = acc + _mm(h1, w1_ref[F:2 * F, :])
    acc = acc + _mm(h2, w1_ref[2 * F:3 * F, :])
    acc = acc + _mm(h3, w1_ref[3 * F:4 * F, :])
    g = jnp.maximum(acc + b1_ref[...], 0.0)
    g_ref[...] = g
    gs_ref[...] = g * d


def _l1_call(x, u1, u2, u3, dinv, w1, b1):
    return pl.pallas_call(
        _l1_body,
        grid=(_GRID,),
        in_specs=[_row_spec(_BLK, F), _part_spec(F), _part_spec(F),
                  _part_spec(F), _row_spec(_BLK, 1),
                  _full_spec(4 * F, H), _full_spec(1, H)],
        out_specs=(_row_spec(_BLK, H), _row_spec(_BLK, H)),
        out_shape=(jax.ShapeDtypeStruct((NP, H), jnp.float32),
                   jax.ShapeDtypeStruct((NP, H), jnp.float32)),
    )(x, u1, u2, u3, dinv, w1, b1)


def _l2_body(g_ref, v1_ref, v2_ref, dinv_ref, ts_ref, tmm_ref, w2_ref, b2_ref,
             wt_ref, bt_ref, wf_ref, bf_ref, out_ref):
    d = dinv_ref[...]
    h1 = (v1_ref[0] + v1_ref[1]) * d
    h2 = (v2_ref[0] + v2_ref[1]) * d
    acc = _mm(g_ref[...], w2_ref[0:H, :])
    acc = acc + _mm(h1, w2_ref[H:2 * H, :])
    acc = acc + _mm(h2, w2_ref[2 * H:3 * H, :])
    z = jnp.maximum(acc + b2_ref[...], 0.0)
    tmin = tmm_ref[0, 0]
    tmax = tmm_ref[0, 1]
    tn = (ts_ref[...] - tmin) / (tmax - tmin + 1e-8)
    te = tn * wt_ref[...] + bt_ref[...]
    out = _mm(z, wf_ref[0:H, :]) + _mm(te, wf_ref[H:2 * H, :]) + bf_ref[...]
    out_ref[...] = out


def _l2_call(g, v1, v2, dinv, ts2, tmm, w2, b2, wt, bt, wf, bf):
    return pl.pallas_call(
        _l2_body,
        grid=(_GRID,),
        in_specs=[_row_spec(_BLK, H), _part_spec(H), _part_spec(H),
                  _row_spec(_BLK, 1), _row_spec(_BLK, 1), _full_spec(1, 2),
                  _full_spec(3 * H, H), _full_spec(1, H),
                  _full_spec(1, H), _full_spec(1, H),
                  _full_spec(2 * H, OUT), _full_spec(1, OUT)],
        out_specs=_row_spec(_BLK, OUT),
        out_shape=jax.ShapeDtypeStruct((NP, OUT), jnp.float32),
    )(g, v1, v2, dinv, ts2, tmm, w2, b2, wt, bt, wf, bf)


def kernel(x, edge_index, time_steps, W1, b1, W2, b2, Wt, bt, Wf, bf):
    src = edge_index[0]
    dst = edge_index[1]
    ones_ch = jnp.ones((CH, F), jnp.float32)
    zeros128 = jnp.zeros((RPT, F), jnp.float32)

    xp = jnp.pad(x, ((0, NP - N), (0, 0)))
    degp = _deg_call(dst, ones_ch, zeros128).reshape(NC, NP, F)
    ts_r = time_steps.astype(jnp.float32).reshape(625, 16)
    dinv, tmm, xs = _prep_call(degp, ts_r, xp)

    u1 = _prop_call(src, dst, xs, zeros128).reshape(NC, NP, F)
    s1 = _scale_call(u1, dinv)
    u2 = _prop_call(src, dst, s1, zeros128).reshape(NC, NP, F)
    s2 = _scale_call(u2, dinv)
    u3 = _prop_call(src, dst, s2, zeros128).reshape(NC, NP, F)

    g, gs = _l1_call(xp, u1, u2, u3, dinv, W1, b1.reshape(1, H))

    v1 = _prop_call(src, dst, gs, zeros128).reshape(NC, NP, F)
    t1 = _scale_call(v1, dinv)
    v2 = _prop_call(src, dst, t1, zeros128).reshape(NC, NP, F)

    ts2 = jnp.pad(time_steps.astype(jnp.float32), (0, NP - N)).reshape(NP, 1)
    out = _l2_call(g, v1, v2, dinv, ts2, tmm, W2, b2.reshape(1, H),
                   Wt, bt.reshape(1, H), Wf, bf.reshape(1, OUT))
    return out[:N]


# speedup vs baseline: 17.2550x; 1.0500x over previous
import functools

import jax
import jax.numpy as jnp
from jax import lax
from jax.experimental import pallas as pl
from jax.experimental.pallas import tpu as pltpu
from jax.experimental.pallas import tpu_sc as plsc

N = 10000
NP = 10240
E = 320000
F = 128
H = 128
OUT = 2

NC = 2
NS = 16
NW = NC * NS
EPW = E // NW
CH = 40
NCH = EPW // CH
NBUF = 5
NOUTER = NCH // NBUF
RPT = NP // NS

_PREC = lax.Precision.HIGHEST


def _prop_body(src_hbm, dst_hbm, s_hbm, z_hbm, out_hbm, *scratch):
    idx_d = scratch[0:NBUF]
    rows = scratch[NBUF:2 * NBUF]
    src_v = scratch[2 * NBUF]
    shared = scratch[2 * NBUF + 1]
    gsem = scratch[2 * NBUF + 2:2 * NBUF + 2 + NBUF]
    isem = scratch[2 * NBUF + 2 + NBUF:2 * NBUF + 2 + 2 * NBUF]

    cid = lax.axis_index("c")
    sid = lax.axis_index("s")
    wid = cid * NS + sid
    base = pl.multiple_of(wid * EPW, EPW)

    pltpu.sync_copy(z_hbm, shared.at[pl.ds(sid * RPT, RPT)])
    pltpu.sync_copy(src_hbm.at[pl.ds(base, EPW)], src_v)
    plsc.subcore_barrier()

    def fire(c0, b):
        pltpu.async_copy(
            dst_hbm.at[pl.ds(base + c0 + b * CH, CH)], idx_d[b], isem[b])
        pltpu.async_copy(
            s_hbm.at[src_v.at[pl.ds(c0 + b * CH, CH)]], rows[b], gsem[b])

    for b in range(NBUF):
        fire(0, b)

    def outer(o, carry):
        c1 = pl.multiple_of((o + 1) * NBUF * CH, NBUF * CH)
        for b in range(NBUF):
            pltpu.make_async_copy(dst_hbm.at[pl.ds(0, CH)], idx_d[b],
                                  isem[b]).wait()
            pltpu.make_async_copy(s_hbm.at[pl.ds(0, CH)], rows[b],
                                  gsem[b]).wait()
            pltpu.sync_copy(rows[b], shared.at[idx_d[b]], add=True)

            @pl.when(o < NOUTER - 1)
            def _():
                fire(c1, b)

        return carry

    lax.fori_loop(0, NOUTER, outer, 0, unroll=False)

    plsc.subcore_barrier()
    off = cid * NP + sid * RPT
    pltpu.sync_copy(shared.at[pl.ds(sid * RPT, RPT)], out_hbm.at[pl.ds(off, RPT)])


@functools.cache
def _sc_mesh():
    return plsc.VectorSubcoreMesh(core_axis_name="c", subcore_axis_name="s",
                                  num_cores=NC, num_subcores=NS)


@functools.cache
def _make_prop():
    return pl.kernel(
        _prop_body,
        out_type=jax.ShapeDtypeStruct((NC * NP, F), jnp.float32),
        mesh=_sc_mesh(),
        scratch_types=(
            [pltpu.VMEM((CH,), jnp.int32) for _ in range(NBUF)]
            + [pltpu.VMEM((CH, F), jnp.float32) for _ in range(NBUF)]
            + [pltpu.VMEM((EPW,), jnp.int32)]
            + [pltpu.VMEM_SHARED((NP, F), jnp.float32)]
            + [pltpu.SemaphoreType.DMA for _ in range(2 * NBUF)]
        ),
    )


def _prop_call(src, dst, s, zeros):
    return _make_prop()(src, dst, s, zeros)


def _deg_body(dst_hbm, ones_hbm, z_hbm, out_hbm, *scratch):
    idx_d = scratch[0:NBUF]
    ones_v = scratch[NBUF]
    shared = scratch[NBUF + 1]
    isem = scratch[NBUF + 2:NBUF + 2 + NBUF]

    cid = lax.axis_index("c")
    sid = lax.axis_index("s")
    wid = cid * NS + sid
    base = pl.multiple_of(wid * EPW, EPW)

    pltpu.sync_copy(z_hbm, shared.at[pl.ds(sid * RPT, RPT)])
    pltpu.sync_copy(ones_hbm, ones_v)
    plsc.subcore_barrier()

    def fire(c0, b):
        pltpu.async_copy(
            dst_hbm.at[pl.ds(base + c0 + b * CH, CH)], idx_d[b], isem[b])

    for b in range(NBUF):
        fire(0, b)

    def outer(o, carry):
        c1 = pl.multiple_of((o + 1) * NBUF * CH, NBUF * CH)
        for b in range(NBUF):
            pltpu.make_async_copy(dst_hbm.at[pl.ds(0, CH)], idx_d[b],
                                  isem[b]).wait()
            pltpu.sync_copy(ones_v, shared.at[idx_d[b]], add=True)

            @pl.when(o < NOUTER - 1)
            def _():
                fire(c1, b)

        return carry

    lax.fori_loop(0, NOUTER, outer, 0, unroll=False)

    plsc.subcore_barrier()
    off = cid * NP + sid * RPT
    pltpu.sync_copy(shared.at[pl.ds(sid * RPT, RPT)], out_hbm.at[pl.ds(off, RPT)])


@functools.cache
def _make_deg():
    return pl.kernel(
        _deg_body,
        out_type=jax.ShapeDtypeStruct((NC * NP,), jnp.float32),
        mesh=_sc_mesh(),
        scratch_types=(
            [pltpu.VMEM((CH,), jnp.int32) for _ in range(NBUF)]
            + [pltpu.VMEM((CH,), jnp.float32)]
            + [pltpu.VMEM_SHARED((NP,), jnp.float32)]
            + [pltpu.SemaphoreType.DMA for _ in range(NBUF)]
        ),
    )


def _deg_call(dst, ones, zeros):
    return _make_deg()(dst, ones, zeros)


def _prep_body(degp_ref, ts_ref, x_ref, dinv_ref, tmm_ref, xs_ref):
    deg = degp_ref[0] + degp_ref[1]
    dinv = jnp.where(deg > 0.0, lax.rsqrt(deg), 0.0)
    dinv_ref[...] = dinv
    ts = ts_ref[...]
    tmm_ref[...] = jnp.concatenate(
        [jnp.min(ts).reshape(1, 1), jnp.max(ts).reshape(1, 1)], axis=1)
    xs_ref[...] = x_ref[...] * dinv


def _prep_call(degp, ts_r, x):
    return pl.pallas_call(
        _prep_body,
        out_shape=(
            jax.ShapeDtypeStruct((NP, 1), jnp.float32),
            jax.ShapeDtypeStruct((1, 2), jnp.float32),
            jax.ShapeDtypeStruct((NP, F), jnp.float32),
        ),
    )(degp, ts_r, x)


_BLK = 512
_GRID = NP // _BLK


def _row_spec(r, c):
    return pl.BlockSpec((r, c), lambda i: (i, 0))


def _part_spec(c):
    return pl.BlockSpec((NC, _BLK, c), lambda i: (0, i, 0))


def _full_spec(r, c):
    return pl.BlockSpec((r, c), lambda i: (0, 0))


def _scale_body(up_ref, dinv_ref, s_ref):
    d = dinv_ref[...]
    s_ref[...] = (up_ref[0] + up_ref[1]) * (d * d)


def _scale_call(up, dinv):
    return pl.pallas_call(
        _scale_body,
        grid=(_GRID,),
        in_specs=[_part_spec(F), _row_spec(_BLK, 1)],
        out_specs=_row_spec(_BLK, F),
        out_shape=jax.ShapeDtypeStruct((NP, F), jnp.float32),
    )(up, dinv)


def _mm(a, w):
    return jnp.dot(a, w, preferred_element_type=jnp.float32, precision=_PREC)


def _l1_body(x_ref, u1_ref, u2_ref, u3_ref, dinv_ref, w1_ref, b1_ref,
             g_ref, gs_ref):
    d = dinv_ref[...]
    h1 = (u1_ref[0] + u1_ref[1]) * d
    h2 = (u2_ref[0] + u2_ref[1]) * d
    h3 = (u3_ref[0] + u3_ref[1]) * d
    acc = _mm(x_ref[...], w1_ref[0:F, :])
    acc = acc + _mm(h1, w1_ref[F:2 * F, :])
    acc = acc + _mm(h2, w1_ref[2 * F:3 * F, :])
    acc = acc + _mm(h3, w1_ref[3 * F:4 * F, :])
    g = jnp.maximum(acc + b1_ref[...], 0.0)
    g_ref[...] = g
    gs_ref[...] = g * d


def _l1_call(x, u1, u2, u3, dinv, w1, b1):
    return pl.pallas_call(
        _l1_body,
        grid=(_GRID,),
        in_specs=[_row_spec(_BLK, F), _part_spec(F), _part_spec(F),
                  _part_spec(F), _row_spec(_BLK, 1),
                  _full_spec(4 * F, H), _full_spec(1, H)],
        out_specs=(_row_spec(_BLK, H), _row_spec(_BLK, H)),
        out_shape=(jax.ShapeDtypeStruct((NP, H), jnp.float32),
                   jax.ShapeDtypeStruct((NP, H), jnp.float32)),
    )(x, u1, u2, u3, dinv, w1, b1)


def _l2_body(g_ref, v1_ref, v2_ref, dinv_ref, ts_ref, tmm_ref, w2_ref, b2_ref,
             wt_ref, bt_ref, wf_ref, bf_ref, out_ref):
    d = dinv_ref[...]
    h1 = (v1_ref[0] + v1_ref[1]) * d
    h2 = (v2_ref[0] + v2_ref[1]) * d
    acc = _mm(g_ref[...], w2_ref[0:H, :])
    acc = acc + _mm(h1, w2_ref[H:2 * H, :])
    acc = acc + _mm(h2, w2_ref[2 * H:3 * H, :])
    z = jnp.maximum(acc + b2_ref[...], 0.0)
    tmin = tmm_ref[0, 0]
    tmax = tmm_ref[0, 1]
    tn = (ts_ref[...] - tmin) / (tmax - tmin + 1e-8)
    te = tn * wt_ref[...] + bt_ref[...]
    out = _mm(z, wf_ref[0:H, :]) + _mm(te, wf_ref[H:2 * H, :]) + bf_ref[...]
    out_ref[...] = out


def _l2_call(g, v1, v2, dinv, ts2, tmm, w2, b2, wt, bt, wf, bf):
    return pl.pallas_call(
        _l2_body,
        grid=(_GRID,),
        in_specs=[_row_spec(_BLK, H), _part_spec(H), _part_spec(H),
                  _row_spec(_BLK, 1), _row_spec(_BLK, 1), _full_spec(1, 2),
                  _full_spec(3 * H, H), _full_spec(1, H),
                  _full_spec(1, H), _full_spec(1, H),
                  _full_spec(2 * H, OUT), _full_spec(1, OUT)],
        out_specs=_row_spec(_BLK, OUT),
        out_shape=jax.ShapeDtypeStruct((NP, OUT), jnp.float32),
    )(g, v1, v2, dinv, ts2, tmm, w2, b2, wt, bt, wf, bf)


def kernel(x, edge_index, time_steps, W1, b1, W2, b2, Wt, bt, Wf, bf):
    src = edge_index[0]
    dst = edge_index[1]
    ones_d = jnp.ones((CH,), jnp.float32)
    zeros_d = jnp.zeros((RPT,), jnp.float32)
    zeros128 = jnp.zeros((RPT, F), jnp.float32)

    xp = jnp.pad(x, ((0, NP - N), (0, 0)))
    degp = _deg_call(dst, ones_d, zeros_d).reshape(NC, NP, 1)
    ts_r = time_steps.astype(jnp.float32).reshape(625, 16)
    dinv, tmm, xs = _prep_call(degp, ts_r, xp)

    u1 = _prop_call(src, dst, xs, zeros128).reshape(NC, NP, F)
    s1 = _scale_call(u1, dinv)
    u2 = _prop_call(src, dst, s1, zeros128).reshape(NC, NP, F)
    s2 = _scale_call(u2, dinv)
    u3 = _prop_call(src, dst, s2, zeros128).reshape(NC, NP, F)

    g, gs = _l1_call(xp, u1, u2, u3, dinv, W1, b1.reshape(1, H))

    v1 = _prop_call(src, dst, gs, zeros128).reshape(NC, NP, F)
    t1 = _scale_call(v1, dinv)
    v2 = _prop_call(src, dst, t1, zeros128).reshape(NC, NP, F)

    ts2 = jnp.pad(time_steps.astype(jnp.float32), (0, NP - N)).reshape(NP, 1)
    out = _l2_call(g, v1, v2, dinv, ts2, tmm, W2, b2.reshape(1, H),
                   Wt, bt.reshape(1, H), Wf, bf.reshape(1, OUT))
    return out[:N]


# speedup vs baseline: 18.5166x; 1.0731x over previous
import functools

import jax
import jax.numpy as jnp
from jax import lax
from jax.experimental import pallas as pl
from jax.experimental.pallas import tpu as pltpu
from jax.experimental.pallas import tpu_sc as plsc

N = 10000
NP = 10240
E = 320000
F = 128
H = 128
OUT = 2

NC = 2
NS = 16
NW = NC * NS
EPW = E // NW
CH = 40
NCH = EPW // CH
NBUF = 5
NOUTER = NCH // NBUF
CHD = 80
NOUTERD = EPW // CHD // NBUF
RPT = NP // NS

_PREC = lax.Precision.HIGHEST


def _prop_body(src_hbm, dst_hbm, s_hbm, z_hbm, out_hbm, *scratch):
    idx_d = scratch[0:NBUF]
    rows = scratch[NBUF:2 * NBUF]
    src_v = scratch[2 * NBUF]
    shared = scratch[2 * NBUF + 1]
    gsem = scratch[2 * NBUF + 2:2 * NBUF + 2 + NBUF]
    isem = scratch[2 * NBUF + 2 + NBUF:2 * NBUF + 2 + 2 * NBUF]

    cid = lax.axis_index("c")
    sid = lax.axis_index("s")
    wid = cid * NS + sid
    base = pl.multiple_of(wid * EPW, EPW)

    pltpu.sync_copy(z_hbm, shared.at[pl.ds(sid * RPT, RPT)])
    pltpu.sync_copy(src_hbm.at[pl.ds(base, EPW)], src_v)
    plsc.subcore_barrier()

    def fire(c0, b):
        pltpu.async_copy(
            dst_hbm.at[pl.ds(base + c0 + b * CH, CH)], idx_d[b], isem[b])
        pltpu.async_copy(
            s_hbm.at[src_v.at[pl.ds(c0 + b * CH, CH)]], rows[b], gsem[b])

    for b in range(NBUF):
        fire(0, b)

    def outer(o, carry):
        c1 = pl.multiple_of((o + 1) * NBUF * CH, NBUF * CH)
        for b in range(NBUF):
            pltpu.make_async_copy(dst_hbm.at[pl.ds(0, CH)], idx_d[b],
                                  isem[b]).wait()
            pltpu.make_async_copy(s_hbm.at[pl.ds(0, CH)], rows[b],
                                  gsem[b]).wait()
            pltpu.sync_copy(rows[b], shared.at[idx_d[b]], add=True)

            @pl.when(o < NOUTER - 1)
            def _():
                fire(c1, b)

        return carry

    lax.fori_loop(0, NOUTER, outer, 0, unroll=False)

    plsc.subcore_barrier()
    off = cid * NP + sid * RPT
    pltpu.sync_copy(shared.at[pl.ds(sid * RPT, RPT)], out_hbm.at[pl.ds(off, RPT)])


@functools.cache
def _sc_mesh():
    return plsc.VectorSubcoreMesh(core_axis_name="c", subcore_axis_name="s",
                                  num_cores=NC, num_subcores=NS)


@functools.cache
def _make_prop():
    return pl.kernel(
        _prop_body,
        out_type=jax.ShapeDtypeStruct((NC * NP, F), jnp.float32),
        mesh=_sc_mesh(),
        scratch_types=(
            [pltpu.VMEM((CH,), jnp.int32) for _ in range(NBUF)]
            + [pltpu.VMEM((CH, F), jnp.float32) for _ in range(NBUF)]
            + [pltpu.VMEM((EPW,), jnp.int32)]
            + [pltpu.VMEM_SHARED((NP, F), jnp.float32)]
            + [pltpu.SemaphoreType.DMA for _ in range(2 * NBUF)]
        ),
    )


def _prop_call(src, dst, s, zeros):
    return _make_prop()(src, dst, s, zeros)


def _deg_body(dst_hbm, ones_hbm, z_hbm, out_hbm, *scratch):
    idx_d = scratch[0:NBUF]
    ones_v = scratch[NBUF]
    shared = scratch[NBUF + 1]
    isem = scratch[NBUF + 2:NBUF + 2 + NBUF]

    cid = lax.axis_index("c")
    sid = lax.axis_index("s")
    wid = cid * NS + sid
    base = pl.multiple_of(wid * EPW, EPW)

    pltpu.sync_copy(z_hbm, shared.at[pl.ds(sid * RPT, RPT)])
    pltpu.sync_copy(ones_hbm, ones_v)
    plsc.subcore_barrier()

    def fire(c0, b):
        pltpu.async_copy(
            dst_hbm.at[pl.ds(base + c0 + b * CHD, CHD)], idx_d[b], isem[b])

    for b in range(NBUF):
        fire(0, b)

    def outer(o, carry):
        c1 = pl.multiple_of((o + 1) * NBUF * CHD, NBUF * CHD)
        for b in range(NBUF):
            pltpu.make_async_copy(dst_hbm.at[pl.ds(0, CHD)], idx_d[b],
                                  isem[b]).wait()
            pltpu.sync_copy(ones_v, shared.at[idx_d[b]], add=True)

            @pl.when(o < NOUTERD - 1)
            def _():
                fire(c1, b)

        return carry

    lax.fori_loop(0, NOUTERD, outer, 0, unroll=False)

    plsc.subcore_barrier()
    off = cid * NP + sid * RPT
    pltpu.sync_copy(shared.at[pl.ds(sid * RPT, RPT)], out_hbm.at[pl.ds(off, RPT)])


@functools.cache
def _make_deg():
    return pl.kernel(
        _deg_body,
        out_type=jax.ShapeDtypeStruct((NC * NP,), jnp.float32),
        mesh=_sc_mesh(),
        scratch_types=(
            [pltpu.VMEM((CHD,), jnp.int32) for _ in range(NBUF)]
            + [pltpu.VMEM((CHD,), jnp.float32)]
            + [pltpu.VMEM_SHARED((NP,), jnp.float32)]
            + [pltpu.SemaphoreType.DMA for _ in range(NBUF)]
        ),
    )


def _deg_call(dst, ones, zeros):
    return _make_deg()(dst, ones, zeros)


def _prep_body(deg0_ref, deg1_ref, ts_ref, x_ref, dinv_ref, tmm_ref, xs_ref):
    deg = deg0_ref[...] + deg1_ref[...]
    dinv1 = jnp.where(deg > 0.0, lax.rsqrt(deg), 0.0)
    dinv = dinv1.reshape(NP, 1)
    dinv_ref[...] = dinv
    ts = ts_ref[...]
    tmm_ref[...] = jnp.concatenate(
        [jnp.min(ts).reshape(1, 1), jnp.max(ts).reshape(1, 1)], axis=1)
    xs_ref[...] = x_ref[...] * dinv


def _prep_call(deg0, deg1, ts_r, x):
    return pl.pallas_call(
        _prep_body,
        out_shape=(
            jax.ShapeDtypeStruct((NP, 1), jnp.float32),
            jax.ShapeDtypeStruct((1, 2), jnp.float32),
            jax.ShapeDtypeStruct((NP, F), jnp.float32),
        ),
    )(deg0, deg1, ts_r, x)


_BLK = 1024
_GRID = NP // _BLK


def _row_spec(r, c):
    return pl.BlockSpec((r, c), lambda i: (i, 0))


def _half_spec(c, half):
    off = half * _GRID
    return pl.BlockSpec((_BLK, c), lambda i, off=off: (i + off, 0))


def _full_spec(r, c):
    return pl.BlockSpec((r, c), lambda i: (0, 0))


def _scale_body(u0_ref, u1_ref, dinv_ref, s_ref):
    d = dinv_ref[...]
    s_ref[...] = (u0_ref[...] + u1_ref[...]) * (d * d)


def _scale_call(up, dinv):
    return pl.pallas_call(
        _scale_body,
        grid=(_GRID,),
        in_specs=[_half_spec(F, 0), _half_spec(F, 1), _row_spec(_BLK, 1)],
        out_specs=_row_spec(_BLK, F),
        out_shape=jax.ShapeDtypeStruct((NP, F), jnp.float32),
    )(up, up, dinv)


def _mm(a, w):
    return jnp.dot(a, w, preferred_element_type=jnp.float32, precision=_PREC)


def _l1_body(x_ref, u1a_ref, u1b_ref, u2a_ref, u2b_ref, u3a_ref, u3b_ref,
             dinv_ref, w1_ref, b1_ref, g_ref, gs_ref):
    d = dinv_ref[...]
    h1 = (u1a_ref[...] + u1b_ref[...]) * d
    h2 = (u2a_ref[...] + u2b_ref[...]) * d
    h3 = (u3a_ref[...] + u3b_ref[...]) * d
    acc = _mm(x_ref[...], w1_ref[0:F, :])
    acc = acc + _mm(h1, w1_ref[F:2 * F, :])
    acc = acc + _mm(h2, w1_ref[2 * F:3 * F, :])
    acc = acc + _mm(h3, w1_ref[3 * F:4 * F, :])
    g = jnp.maximum(acc + b1_ref[...], 0.0)
    g_ref[...] = g
    gs_ref[...] = g * d


def _l1_call(x, u1, u2, u3, dinv, w1, b1):
    return pl.pallas_call(
        _l1_body,
        grid=(_GRID,),
        in_specs=[_row_spec(_BLK, F),
                  _half_spec(F, 0), _half_spec(F, 1),
                  _half_spec(F, 0), _half_spec(F, 1),
                  _half_spec(F, 0), _half_spec(F, 1),
                  _row_spec(_BLK, 1),
                  _full_spec(4 * F, H), _full_spec(1, H)],
        out_specs=(_row_spec(_BLK, H), _row_spec(_BLK, H)),
        out_shape=(jax.ShapeDtypeStruct((NP, H), jnp.float32),
                   jax.ShapeDtypeStruct((NP, H), jnp.float32)),
    )(x, u1, u1, u2, u2, u3, u3, dinv, w1, b1)


def _l2_body(g_ref, v1a_ref, v1b_ref, v2a_ref, v2b_ref, dinv_ref, ts_ref,
             tmm_ref, w2_ref, b2_ref, wt_ref, bt_ref, wf_ref, bf_ref, out_ref):
    d = dinv_ref[...]
    h1 = (v1a_ref[...] + v1b_ref[...]) * d
    h2 = (v2a_ref[...] + v2b_ref[...]) * d
    acc = _mm(g_ref[...], w2_ref[0:H, :])
    acc = acc + _mm(h1, w2_ref[H:2 * H, :])
    acc = acc + _mm(h2, w2_ref[2 * H:3 * H, :])
    z = jnp.maximum(acc + b2_ref[...], 0.0)
    tmin = tmm_ref[0, 0]
    tmax = tmm_ref[0, 1]
    tn = (ts_ref[...] - tmin) / (tmax - tmin + 1e-8)
    te = tn * wt_ref[...] + bt_ref[...]
    out = _mm(z, wf_ref[0:H, :]) + _mm(te, wf_ref[H:2 * H, :]) + bf_ref[...]
    out_ref[...] = out


def _l2_call(g, v1, v2, dinv, ts2, tmm, w2, b2, wt, bt, wf, bf):
    return pl.pallas_call(
        _l2_body,
        grid=(_GRID,),
        in_specs=[_row_spec(_BLK, H),
                  _half_spec(H, 0), _half_spec(H, 1),
                  _half_spec(H, 0), _half_spec(H, 1),
                  _row_spec(_BLK, 1), _row_spec(_BLK, 1), _full_spec(1, 2),
                  _full_spec(3 * H, H), _full_spec(1, H),
                  _full_spec(1, H), _full_spec(1, H),
                  _full_spec(2 * H, OUT), _full_spec(1, OUT)],
        out_specs=_row_spec(_BLK, OUT),
        out_shape=jax.ShapeDtypeStruct((NP, OUT), jnp.float32),
    )(g, v1, v1, v2, v2, dinv, ts2, tmm, w2, b2, wt, bt, wf, bf)


def kernel(x, edge_index, time_steps, W1, b1, W2, b2, Wt, bt, Wf, bf):
    src = edge_index[0]
    dst = edge_index[1]
    ones_d = jnp.ones((CHD,), jnp.float32)
    zeros_d = jnp.zeros((RPT,), jnp.float32)
    zeros128 = jnp.zeros((RPT, F), jnp.float32)

    xp = jnp.pad(x, ((0, NP - N), (0, 0)))
    degp = _deg_call(dst, ones_d, zeros_d)
    ts_r = time_steps.astype(jnp.float32).reshape(625, 16)
    dinv, tmm, xs = _prep_call(degp[:NP], degp[NP:], ts_r, xp)

    u1 = _prop_call(src, dst, xs, zeros128)
    s1 = _scale_call(u1, dinv)
    u2 = _prop_call(src, dst, s1, zeros128)
    s2 = _scale_call(u2, dinv)
    u3 = _prop_call(src, dst, s2, zeros128)

    g, gs = _l1_call(xp, u1, u2, u3, dinv, W1, b1.reshape(1, H))

    v1 = _prop_call(src, dst, gs, zeros128)
    t1 = _scale_call(v1, dinv)
    v2 = _prop_call(src, dst, t1, zeros128)

    ts2 = jnp.pad(time_steps.astype(jnp.float32), (0, NP - N)).reshape(NP, 1)
    out = _l2_call(g, v1, v2, dinv, ts2, tmm, W2, b2.reshape(1, H),
                   Wt, bt.reshape(1, H), Wf, bf.reshape(1, OUT))
    return out[:N]


# speedup vs baseline: 19.2892x; 1.0417x over previous
import functools

import jax
import jax.numpy as jnp
from jax import lax
from jax.experimental import pallas as pl
from jax.experimental.pallas import tpu as pltpu
from jax.experimental.pallas import tpu_sc as plsc

N = 10000
NP = 10240
E = 320000
F = 128
H = 128
OUT = 2

NC = 2
NS = 16
NW = NC * NS
EPW = E // NW
CH = 40
NCH = EPW // CH
NBUF = 5
NOUTER = NCH // NBUF
CHD = 80
NOUTERD = EPW // CHD // NBUF
RPT = NP // NS

_PREC = lax.Precision.DEFAULT


def _prop_body(src_hbm, dst_hbm, s_hbm, z_hbm, out_hbm, *scratch):
    idx_d = scratch[0:NBUF]
    rows = scratch[NBUF:2 * NBUF]
    src_v = scratch[2 * NBUF]
    shared = scratch[2 * NBUF + 1]
    gsem = scratch[2 * NBUF + 2:2 * NBUF + 2 + NBUF]
    isem = scratch[2 * NBUF + 2 + NBUF:2 * NBUF + 2 + 2 * NBUF]

    cid = lax.axis_index("c")
    sid = lax.axis_index("s")
    wid = cid * NS + sid
    base = pl.multiple_of(wid * EPW, EPW)

    pltpu.sync_copy(z_hbm, shared.at[pl.ds(sid * RPT, RPT)])
    pltpu.sync_copy(src_hbm.at[pl.ds(base, EPW)], src_v)
    plsc.subcore_barrier()

    def fire(c0, b):
        pltpu.async_copy(
            dst_hbm.at[pl.ds(base + c0 + b * CH, CH)], idx_d[b], isem[b])
        pltpu.async_copy(
            s_hbm.at[src_v.at[pl.ds(c0 + b * CH, CH)]], rows[b], gsem[b])

    for b in range(NBUF):
        fire(0, b)

    def outer(o, carry):
        c1 = pl.multiple_of((o + 1) * NBUF * CH, NBUF * CH)
        for b in range(NBUF):
            pltpu.make_async_copy(dst_hbm.at[pl.ds(0, CH)], idx_d[b],
                                  isem[b]).wait()
            pltpu.make_async_copy(s_hbm.at[pl.ds(0, CH)], rows[b],
                                  gsem[b]).wait()
            pltpu.sync_copy(rows[b], shared.at[idx_d[b]], add=True)

            @pl.when(o < NOUTER - 1)
            def _():
                fire(c1, b)

        return carry

    lax.fori_loop(0, NOUTER, outer, 0, unroll=False)

    plsc.subcore_barrier()
    off = cid * NP + sid * RPT
    pltpu.sync_copy(shared.at[pl.ds(sid * RPT, RPT)], out_hbm.at[pl.ds(off, RPT)])


@functools.cache
def _sc_mesh():
    return plsc.VectorSubcoreMesh(core_axis_name="c", subcore_axis_name="s",
                                  num_cores=NC, num_subcores=NS)


@functools.cache
def _make_prop():
    return pl.kernel(
        _prop_body,
        out_type=jax.ShapeDtypeStruct((NC * NP, F), jnp.float32),
        mesh=_sc_mesh(),
        scratch_types=(
            [pltpu.VMEM((CH,), jnp.int32) for _ in range(NBUF)]
            + [pltpu.VMEM((CH, F), jnp.float32) for _ in range(NBUF)]
            + [pltpu.VMEM((EPW,), jnp.int32)]
            + [pltpu.VMEM_SHARED((NP, F), jnp.float32)]
            + [pltpu.SemaphoreType.DMA for _ in range(2 * NBUF)]
        ),
    )


def _prop_call(src, dst, s, zeros):
    return _make_prop()(src, dst, s, zeros)


def _deg_body(dst_hbm, ones_hbm, z_hbm, out_hbm, *scratch):
    idx_d = scratch[0:NBUF]
    ones_v = scratch[NBUF]
    shared = scratch[NBUF + 1]
    isem = scratch[NBUF + 2:NBUF + 2 + NBUF]

    cid = lax.axis_index("c")
    sid = lax.axis_index("s")
    wid = cid * NS + sid
    base = pl.multiple_of(wid * EPW, EPW)

    pltpu.sync_copy(z_hbm, shared.at[pl.ds(sid * RPT, RPT)])
    pltpu.sync_copy(ones_hbm, ones_v)
    plsc.subcore_barrier()

    def fire(c0, b):
        pltpu.async_copy(
            dst_hbm.at[pl.ds(base + c0 + b * CHD, CHD)], idx_d[b], isem[b])

    for b in range(NBUF):
        fire(0, b)

    def outer(o, carry):
        c1 = pl.multiple_of((o + 1) * NBUF * CHD, NBUF * CHD)
        for b in range(NBUF):
            pltpu.make_async_copy(dst_hbm.at[pl.ds(0, CHD)], idx_d[b],
                                  isem[b]).wait()
            pltpu.sync_copy(ones_v, shared.at[idx_d[b]], add=True)

            @pl.when(o < NOUTERD - 1)
            def _():
                fire(c1, b)

        return carry

    lax.fori_loop(0, NOUTERD, outer, 0, unroll=False)

    plsc.subcore_barrier()
    off = cid * NP + sid * RPT
    pltpu.sync_copy(shared.at[pl.ds(sid * RPT, RPT)], out_hbm.at[pl.ds(off, RPT)])


@functools.cache
def _make_deg():
    return pl.kernel(
        _deg_body,
        out_type=jax.ShapeDtypeStruct((NC * NP,), jnp.float32),
        mesh=_sc_mesh(),
        scratch_types=(
            [pltpu.VMEM((CHD,), jnp.int32) for _ in range(NBUF)]
            + [pltpu.VMEM((CHD,), jnp.float32)]
            + [pltpu.VMEM_SHARED((NP,), jnp.float32)]
            + [pltpu.SemaphoreType.DMA for _ in range(NBUF)]
        ),
    )


def _deg_call(dst, ones, zeros):
    return _make_deg()(dst, ones, zeros)


def _prep_body(deg0_ref, deg1_ref, ts_ref, x_ref, dinv_ref, tmm_ref, xs_ref):
    deg = deg0_ref[...] + deg1_ref[...]
    dinv1 = jnp.where(deg > 0.0, lax.rsqrt(deg), 0.0)
    dinv = dinv1.reshape(NP, 1)
    dinv_ref[...] = dinv
    ts = ts_ref[...]
    tmm_ref[...] = jnp.concatenate(
        [jnp.min(ts).reshape(1, 1), jnp.max(ts).reshape(1, 1)], axis=1)
    xs_ref[...] = x_ref[...] * dinv


def _prep_call(deg0, deg1, ts_r, x):
    return pl.pallas_call(
        _prep_body,
        out_shape=(
            jax.ShapeDtypeStruct((NP, 1), jnp.float32),
            jax.ShapeDtypeStruct((1, 2), jnp.float32),
            jax.ShapeDtypeStruct((NP, F), jnp.float32),
        ),
    )(deg0, deg1, ts_r, x)


_BLK = 1024
_GRID = NP // _BLK


def _row_spec(r, c):
    return pl.BlockSpec((r, c), lambda i: (i, 0))


def _half_spec(c, half):
    off = half * _GRID
    return pl.BlockSpec((_BLK, c), lambda i, off=off: (i + off, 0))


def _full_spec(r, c):
    return pl.BlockSpec((r, c), lambda i: (0, 0))


def _scale_body(u0_ref, u1_ref, dinv_ref, s_ref):
    d = dinv_ref[...]
    s_ref[...] = (u0_ref[...] + u1_ref[...]) * (d * d)


def _scale_call(up, dinv):
    return pl.pallas_call(
        _scale_body,
        grid=(_GRID,),
        in_specs=[_half_spec(F, 0), _half_spec(F, 1), _row_spec(_BLK, 1)],
        out_specs=_row_spec(_BLK, F),
        out_shape=jax.ShapeDtypeStruct((NP, F), jnp.float32),
    )(up, up, dinv)


def _mm(a, w):
    return jnp.dot(a, w, preferred_element_type=jnp.float32, precision=_PREC)


def _l1_body(x_ref, u1a_ref, u1b_ref, u2a_ref, u2b_ref, u3a_ref, u3b_ref,
             dinv_ref, w1_ref, b1_ref, g_ref, gs_ref):
    d = dinv_ref[...]
    h1 = (u1a_ref[...] + u1b_ref[...]) * d
    h2 = (u2a_ref[...] + u2b_ref[...]) * d
    h3 = (u3a_ref[...] + u3b_ref[...]) * d
    acc = _mm(x_ref[...], w1_ref[0:F, :])
    acc = acc + _mm(h1, w1_ref[F:2 * F, :])
    acc = acc + _mm(h2, w1_ref[2 * F:3 * F, :])
    acc = acc + _mm(h3, w1_ref[3 * F:4 * F, :])
    g = jnp.maximum(acc + b1_ref[...], 0.0)
    g_ref[...] = g
    gs_ref[...] = g * d


def _l1_call(x, u1, u2, u3, dinv, w1, b1):
    return pl.pallas_call(
        _l1_body,
        grid=(_GRID,),
        in_specs=[_row_spec(_BLK, F),
                  _half_spec(F, 0), _half_spec(F, 1),
                  _half_spec(F, 0), _half_spec(F, 1),
                  _half_spec(F, 0), _half_spec(F, 1),
                  _row_spec(_BLK, 1),
                  _full_spec(4 * F, H), _full_spec(1, H)],
        out_specs=(_row_spec(_BLK, H), _row_spec(_BLK, H)),
        out_shape=(jax.ShapeDtypeStruct((NP, H), jnp.float32),
                   jax.ShapeDtypeStruct((NP, H), jnp.float32)),
    )(x, u1, u1, u2, u2, u3, u3, dinv, w1, b1)


def _l2_body(g_ref, v1a_ref, v1b_ref, v2a_ref, v2b_ref, dinv_ref, ts_ref,
             tmm_ref, w2_ref, b2_ref, wt_ref, bt_ref, wf_ref, bf_ref, out_ref):
    d = dinv_ref[...]
    h1 = (v1a_ref[...] + v1b_ref[...]) * d
    h2 = (v2a_ref[...] + v2b_ref[...]) * d
    acc = _mm(g_ref[...], w2_ref[0:H, :])
    acc = acc + _mm(h1, w2_ref[H:2 * H, :])
    acc = acc + _mm(h2, w2_ref[2 * H:3 * H, :])
    z = jnp.maximum(acc + b2_ref[...], 0.0)
    tmin = tmm_ref[0, 0]
    tmax = tmm_ref[0, 1]
    tn = (ts_ref[...] - tmin) / (tmax - tmin + 1e-8)
    te = tn * wt_ref[...] + bt_ref[...]
    out = _mm(z, wf_ref[0:H, :]) + _mm(te, wf_ref[H:2 * H, :]) + bf_ref[...]
    out_ref[...] = out


def _l2_call(g, v1, v2, dinv, ts2, tmm, w2, b2, wt, bt, wf, bf):
    return pl.pallas_call(
        _l2_body,
        grid=(_GRID,),
        in_specs=[_row_spec(_BLK, H),
                  _half_spec(H, 0), _half_spec(H, 1),
                  _half_spec(H, 0), _half_spec(H, 1),
                  _row_spec(_BLK, 1), _row_spec(_BLK, 1), _full_spec(1, 2),
                  _full_spec(3 * H, H), _full_spec(1, H),
                  _full_spec(1, H), _full_spec(1, H),
                  _full_spec(2 * H, OUT), _full_spec(1, OUT)],
        out_specs=_row_spec(_BLK, OUT),
        out_shape=jax.ShapeDtypeStruct((NP, OUT), jnp.float32),
    )(g, v1, v1, v2, v2, dinv, ts2, tmm, w2, b2, wt, bt, wf, bf)


def kernel(x, edge_index, time_steps, W1, b1, W2, b2, Wt, bt, Wf, bf):
    src = edge_index[0]
    dst = edge_index[1]
    ones_d = jnp.ones((CHD,), jnp.float32)
    zeros_d = jnp.zeros((RPT,), jnp.float32)
    zeros128 = jnp.zeros((RPT, F), jnp.float32)

    xp = jnp.pad(x, ((0, NP - N), (0, 0)))
    degp = _deg_call(dst, ones_d, zeros_d)
    ts_r = time_steps.astype(jnp.float32).reshape(625, 16)
    dinv, tmm, xs = _prep_call(degp[:NP], degp[NP:], ts_r, xp)

    u1 = _prop_call(src, dst, xs, zeros128)
    s1 = _scale_call(u1, dinv)
    u2 = _prop_call(src, dst, s1, zeros128)
    s2 = _scale_call(u2, dinv)
    u3 = _prop_call(src, dst, s2, zeros128)

    g, gs = _l1_call(xp, u1, u2, u3, dinv, W1, b1.reshape(1, H))

    v1 = _prop_call(src, dst, gs, zeros128)
    t1 = _scale_call(v1, dinv)
    v2 = _prop_call(src, dst, t1, zeros128)

    ts2 = jnp.pad(time_steps.astype(jnp.float32), (0, NP - N)).reshape(NP, 1)
    out = _l2_call(g, v1, v2, dinv, ts2, tmm, W2, b2.reshape(1, H),
                   Wt, bt.reshape(1, H), Wf, bf.reshape(1, OUT))
    return out[:N]
